# baseline (speedup 1.0000x reference)
import functools

import jax
import jax.numpy as jnp
from jax import lax
from jax.experimental import pallas as pl
from jax.experimental.pallas import tpu as pltpu
from jax.experimental.pallas import tpu_sc as plsc

_B = 16384
_D_NUM = 13
_N_CAT = 26
_VOCAB = 100000
_D = 64
_NNUM = 1 + _D_NUM
_NROW = _NNUM + _N_CAT

_NC, _NS = 2, 16
_NW = _NC * _NS
_PER_W = _B // _NW
_SB = 16
_NCHUNK = _PER_W // _SB
_GROWS = 104
_GPC = (_SB * _N_CAT) // _GROWS
_IDX_ROWS = _B * _N_CAT // _GROWS


def _body(idx_hbm, tab_hbm, xnum_hbm, w_hbm, bn_hbm, bc_hbm, out_hbm,
          idx_res, xnum_res, w_res, bn_res, bc_res, cat_stage, stage, sem):
    wid = lax.axis_index("s") * _NC + lax.axis_index("c")

    irows = _PER_W * _N_CAT // _GROWS
    pltpu.sync_copy(idx_hbm.at[pl.ds(wid * irows, irows), :], idx_res)
    pltpu.sync_copy(xnum_hbm.at[pl.ds(wid * _PER_W, _PER_W), :], xnum_res)
    pltpu.sync_copy(w_hbm, w_res)
    pltpu.sync_copy(bn_hbm, bn_res)
    pltpu.sync_copy(bc_hbm, bc_res)

    def chunk_body(c, carry):
        descs = [
            pltpu.async_copy(
                tab_hbm.at[idx_res.at[c * _GPC + gi]],
                cat_stage.at[pl.ds(gi * _GROWS, _GROWS), :],
                sem)
            for gi in range(_GPC)
        ]
        for d in descs:
            d.wait()

        def row_body(b_off, carry2):
            srow = b_off * _NROW
            crow = b_off * _N_CAT
            bloc = c * _SB + b_off
            xv = xnum_res[bloc, :]
            for j in range(_NNUM):
                x = xv[j]
                for ch in range(4):
                    s = pl.ds(ch * 16, 16)
                    stage[srow + j, s] = w_res[j, s] * x + bn_res[j, s]
            for i in range(_N_CAT):
                for ch in range(4):
                    s = pl.ds(ch * 16, 16)
                    stage[srow + _NNUM + i, s] = (
                        cat_stage[crow + i, s] + bc_res[i, s])
            return carry2

        lax.fori_loop(0, _SB, row_body, 0)

        base = (wid * _PER_W + c * _SB) * _NROW
        pltpu.sync_copy(stage, out_hbm.at[pl.ds(base, _SB * _NROW), :])
        return carry

    lax.fori_loop(0, _NCHUNK, chunk_body, 0)


@jax.jit
def _tokenize(idx, tab, xnf, weight, bn, bc):
    mesh = plsc.VectorSubcoreMesh(
        core_axis_name="c", subcore_axis_name="s",
        num_cores=_NC, num_subcores=_NS)
    f = pl.kernel(
        _body,
        out_type=jax.ShapeDtypeStruct((_B * _NROW, _D), jnp.float32),
        mesh=mesh,
        scratch_types=[
            pltpu.VMEM((_PER_W * _N_CAT // _GROWS, _GROWS), jnp.int32),
            pltpu.VMEM((_PER_W, 16), jnp.float32),
            pltpu.VMEM((_NNUM, _D), jnp.float32),
            pltpu.VMEM((_NNUM, _D), jnp.float32),
            pltpu.VMEM((_N_CAT, _D), jnp.float32),
            pltpu.VMEM((_SB * _N_CAT, _D), jnp.float32),
            pltpu.VMEM((_SB * _NROW, _D), jnp.float32),
            pltpu.SemaphoreType.DMA,
        ],
        compiler_params=pltpu.CompilerParams(use_tc_tiling_on_sc=False),
    )
    return f(idx, tab, xnf, weight, bn, bc)


def kernel(x_num, x_cat, emb_tables, weight, bias):
    x_cat = x_cat.astype(jnp.int32)
    idx = x_cat.T + (jnp.arange(_N_CAT, dtype=jnp.int32) * _VOCAB)[None, :]
    idx = idx.reshape(_IDX_ROWS, _GROWS)
    tab = emb_tables.reshape(_N_CAT * _VOCAB, _D)
    xnf = jnp.concatenate(
        [jnp.ones((_B, 1), x_num.dtype), x_num,
         jnp.zeros((_B, 2), x_num.dtype)], axis=1)
    bn = jnp.concatenate(
        [jnp.zeros((1, _D), bias.dtype), bias[:_D_NUM]], axis=0)
    bc = bias[_D_NUM:]
    out = _tokenize(idx, tab, xnf, weight, bn, bc)
    return out.reshape(_B, _NROW, _D)


# speedup vs baseline: 1.0012x; 1.0012x over previous
import functools

import jax
import jax.numpy as jnp
from jax import lax
from jax.experimental import pallas as pl
from jax.experimental.pallas import tpu as pltpu
from jax.experimental.pallas import tpu_sc as plsc

_B = 16384
_D_NUM = 13
_N_CAT = 26
_VOCAB = 100000
_D = 64
_NNUM = 1 + _D_NUM
_NROW = _NNUM + _N_CAT

_NC, _NS = 2, 16
_NW = _NC * _NS
_PER_W = _B // _NW
_SB = 16
_NCHUNK = _PER_W // _SB
_GROWS = 104
_GPC = (_SB * _N_CAT) // _GROWS
_IDX_ROWS = _B * _N_CAT // _GROWS


def _body(idx_hbm, tab_hbm, xnum_hbm, w_hbm, bn_hbm, bc_hbm, out_hbm,
          idx_res, xnum_res, w_res, bn_res, bc_res, cat_stage, stage, sem):
    wid = lax.axis_index("s") * _NC + lax.axis_index("c")

    irows = _PER_W * _N_CAT // _GROWS
    pltpu.sync_copy(idx_hbm.at[pl.ds(wid * irows, irows), :], idx_res)
    pltpu.sync_copy(xnum_hbm.at[pl.ds(wid * _PER_W, _PER_W), :], xnum_res)
    pltpu.sync_copy(w_hbm, w_res)
    pltpu.sync_copy(bn_hbm, bn_res)
    pltpu.sync_copy(bc_hbm, bc_res)

    def chunk_body(c, carry):
        descs = [
            pltpu.async_copy(
                tab_hbm.at[idx_res.at[c * _GPC + gi]],
                cat_stage.at[pl.ds(gi * _GROWS, _GROWS), :],
                sem)
            for gi in range(_GPC)
        ]
        for d in descs:
            d.wait()

        def row_body(b_off, carry2):
            crow = b_off * _N_CAT
            bloc = c * _SB + b_off
            xv = xnum_res[bloc, :]
            for j in range(_NNUM):
                x = xv[j]
                for ch in range(4):
                    s = pl.ds(ch * 16, 16)
                    stage[b_off, j, s] = w_res[j, s] * x + bn_res[j, s]
            for i in range(_N_CAT):
                for ch in range(4):
                    s = pl.ds(ch * 16, 16)
                    stage[b_off, _NNUM + i, s] = (
                        cat_stage[crow + i, s] + bc_res[i, s])
            return carry2

        lax.fori_loop(0, _SB, row_body, 0)

        base = wid * _PER_W + c * _SB
        pltpu.sync_copy(stage, out_hbm.at[pl.ds(base, _SB)])
        return carry

    lax.fori_loop(0, _NCHUNK, chunk_body, 0)


@jax.jit
def _tokenize(idx, tab, xnf, weight, bn, bc):
    mesh = plsc.VectorSubcoreMesh(
        core_axis_name="c", subcore_axis_name="s",
        num_cores=_NC, num_subcores=_NS)
    f = pl.kernel(
        _body,
        out_type=jax.ShapeDtypeStruct((_B, _NROW, _D), jnp.float32),
        mesh=mesh,
        scratch_types=[
            pltpu.VMEM((_PER_W * _N_CAT // _GROWS, _GROWS), jnp.int32),
            pltpu.VMEM((_PER_W, 16), jnp.float32),
            pltpu.VMEM((_NNUM, _D), jnp.float32),
            pltpu.VMEM((_NNUM, _D), jnp.float32),
            pltpu.VMEM((_N_CAT, _D), jnp.float32),
            pltpu.VMEM((_SB * _N_CAT, _D), jnp.float32),
            pltpu.VMEM((_SB, _NROW, _D), jnp.float32),
            pltpu.SemaphoreType.DMA,
        ],
        compiler_params=pltpu.CompilerParams(use_tc_tiling_on_sc=False),
    )
    return f(idx, tab, xnf, weight, bn, bc)


def kernel(x_num, x_cat, emb_tables, weight, bias):
    x_cat = x_cat.astype(jnp.int32)
    idx = x_cat.T + (jnp.arange(_N_CAT, dtype=jnp.int32) * _VOCAB)[None, :]
    idx = idx.reshape(_IDX_ROWS, _GROWS)
    tab = emb_tables.reshape(_N_CAT * _VOCAB, _D)
    xnf = jnp.concatenate(
        [jnp.ones((_B, 1), x_num.dtype), x_num,
         jnp.zeros((_B, 2), x_num.dtype)], axis=1)
    bn = jnp.concatenate(
        [jnp.zeros((1, _D), bias.dtype), bias[:_D_NUM]], axis=0)
    bc = bias[_D_NUM:]
    return _tokenize(idx, tab, xnf, weight, bn, bc)


# speedup vs baseline: 1.0097x; 1.0084x over previous
import functools

import jax
import jax.numpy as jnp
from jax import lax
from jax.experimental import pallas as pl
from jax.experimental.pallas import tpu as pltpu
from jax.experimental.pallas import tpu_sc as plsc

_B = 16384
_D_NUM = 13
_N_CAT = 26
_VOCAB = 100000
_D = 64
_NNUM = 1 + _D_NUM
_NROW = _NNUM + _N_CAT

_NC, _NS = 2, 16
_NW = _NC * _NS
_PER_W = _B // _NW
_SB = 8
_NCHUNK = _PER_W // _SB
_GROWS = 104
_GPC = (_SB * _N_CAT) // _GROWS
_IDX_ROWS = _B * _N_CAT // _GROWS


def _body(idx_hbm, tab_hbm, xnum_hbm, par_hbm, w_hbm, bn_hbm, bc_hbm,
          out_hbm,
          idx_res, xnum_res, par_res, w_res, bn_res, bc_res,
          cat_stage, stage, sem):
    wid = lax.axis_index("s") * _NC + lax.axis_index("c")

    irows = _PER_W * _N_CAT // _GROWS
    pltpu.sync_copy(idx_hbm.at[pl.ds(wid * irows, irows), :], idx_res)
    pltpu.sync_copy(xnum_hbm.at[pl.ds(wid * (_PER_W // 8), _PER_W // 8), :],
                    xnum_res)
    pltpu.sync_copy(par_hbm.at[pl.ds(wid * (_PER_W // 4), _PER_W // 4), :],
                    par_res)
    pltpu.sync_copy(w_hbm, w_res)
    pltpu.sync_copy(bn_hbm, bn_res)
    pltpu.sync_copy(bc_hbm, bc_res)

    def chunk_body(c, carry):
        descs = [
            pltpu.async_copy(
                tab_hbm.at[idx_res.at[c * _GPC + gi]],
                cat_stage.at[pl.ds(gi * _GROWS, _GROWS), :],
                sem)
            for gi in range(_GPC)
        ]
        for d in descs:
            d.wait()

        def row_body(b_off, carry2):
            crow = b_off * _N_CAT
            bloc = c * _SB + b_off
            xv = xnum_res[bloc >> 3, pl.ds((bloc & 7) * 16, 16)]
            for j in range(_NNUM):
                x = xv[j]
                for ch in range(4):
                    s = pl.ds(ch * 16, 16)
                    stage[b_off, j, s] = w_res[j, s] * x + bn_res[j, s]
            pr = bloc >> 2
            pc = (bloc & 3) * 32
            pv0 = par_res[pr, pl.ds(pc, 16)]
            pv1 = par_res[pr, pl.ds(pc + 16, 16)]
            for i in range(_N_CAT):
                po = pv0[i] if i < 16 else pv1[i - 16]
                for ch in range(4):
                    stage[b_off, _NNUM + i, pl.ds(ch * 16, 16)] = (
                        cat_stage[crow + i, pl.ds(po + ch * 16, 16)]
                        + bc_res[i, pl.ds(ch * 16, 16)])
            return carry2

        lax.fori_loop(0, _SB, row_body, 0)

        base = wid * _PER_W + c * _SB
        pltpu.sync_copy(stage, out_hbm.at[pl.ds(base, _SB)])
        return carry

    lax.fori_loop(0, _NCHUNK, chunk_body, 0)


@jax.jit
def _tokenize(idx, tab, xnf, par, weight, bn, bc):
    mesh = plsc.VectorSubcoreMesh(
        core_axis_name="c", subcore_axis_name="s",
        num_cores=_NC, num_subcores=_NS)
    f = pl.kernel(
        _body,
        out_type=jax.ShapeDtypeStruct((_B, _NROW, _D), jnp.float32),
        mesh=mesh,
        scratch_types=[
            pltpu.VMEM((_PER_W * _N_CAT // _GROWS, _GROWS), jnp.int32),
            pltpu.VMEM((_PER_W // 8, 128), jnp.float32),
            pltpu.VMEM((_PER_W // 4, 128), jnp.int32),
            pltpu.VMEM((_NNUM, _D), jnp.float32),
            pltpu.VMEM((_NNUM, _D), jnp.float32),
            pltpu.VMEM((_N_CAT, _D), jnp.float32),
            pltpu.VMEM((_SB * _N_CAT, 128), jnp.float32),
            pltpu.VMEM((_SB, _NROW, _D), jnp.float32),
            pltpu.SemaphoreType.DMA,
        ],
        compiler_params=pltpu.CompilerParams(use_tc_tiling_on_sc=True),
    )
    return f(idx, tab, xnf, par, weight, bn, bc)


def kernel(x_num, x_cat, emb_tables, weight, bias):
    x_cat = x_cat.astype(jnp.int32)
    g = x_cat.T + (jnp.arange(_N_CAT, dtype=jnp.int32) * _VOCAB)[None, :]
    idx = (g >> 1).reshape(_IDX_ROWS, _GROWS)
    par = jnp.pad((g & 1) << 6, ((0, 0), (0, 6)))
    par = par.reshape(_B // 4, 128)
    tab = emb_tables.reshape(_N_CAT * _VOCAB // 2, 128)
    xnf = jnp.concatenate(
        [jnp.ones((_B, 1), x_num.dtype), x_num,
         jnp.zeros((_B, 2), x_num.dtype)], axis=1)
    xnf = xnf.reshape(_B // 8, 128)
    bn = jnp.concatenate(
        [jnp.zeros((1, _D), bias.dtype), bias[:_D_NUM]], axis=0)
    bc = bias[_D_NUM:]
    return _tokenize(idx, tab, xnf, par, weight, bn, bc)
